# baseline (speedup 1.0000x reference)
import jax
import jax.numpy as jnp
from jax import lax
from jax.experimental import pallas as pl
from jax.experimental.pallas import tpu as pltpu
from jax.experimental.pallas import tpu_sc as plsc

_B, _C, _H, _W = 16, 3, 512, 512
_N = _B * _H * _W
_BLK_H = 64

_NW = 32
_PPW = _N // _NW
_CH = 2048
_NCHUNK = _PPW // _CH
_KG = _CH // 128


def _stage_a_body(w_ref, i0_ref, i1_ref, sel_ref,
                  wa_ref, wb_ref, wc_ref, wd_ref):
    b = pl.program_id(0)
    hb = pl.program_id(1)
    X = w_ref[0, 0]
    Y = w_ref[0, 1]
    T = w_ref[0, 2]
    xx = lax.broadcasted_iota(jnp.int32, (_BLK_H, _W), 1).astype(jnp.float32)
    yyi = lax.broadcasted_iota(jnp.int32, (_BLK_H, _W), 0) + hb * _BLK_H
    yy = yyi.astype(jnp.float32)
    sm = jnp.where(jnp.abs(T) >= 1e-07, jnp.float32(0.0), jnp.float32(1e-06))
    Tt = T + sm
    v1 = X / Tt
    v2 = Y / Tt
    vgx = xx + (v1 - xx)
    vgy = yy + (v2 - yy)
    x0i = jnp.floor(vgx).astype(jnp.int32)
    y0i = jnp.floor(vgy).astype(jnp.int32)
    x0 = jnp.clip(x0i, 0, _W - 1)
    x1 = jnp.clip(x0i + 1, 0, _W - 1)
    y0 = jnp.clip(y0i, 0, _H - 1)
    y1 = jnp.clip(y0i + 1, 0, _H - 1)
    x0f = x0.astype(jnp.float32)
    x1f = x1.astype(jnp.float32)
    y0f = y0.astype(jnp.float32)
    y1f = y1.astype(jnp.float32)
    Xa = x1f - vgx
    Xc = vgx - x0f
    Ya = y1f - vgy
    Yb = vgy - y0f
    wa_ref[0] = Xa * Ya
    wb_ref[0] = Xa * Yb
    wc_ref[0] = Xc * Ya
    wd_ref[0] = Xc * Yb
    base = b * (_H * _W)
    i0_ref[0] = (base + y0 * _W) + x0
    i1_ref[0] = (base + y1 * _W) + x0
    sel_ref[0] = x1 - x0


def _stage_a(warped):
    n_hb = _H // _BLK_H
    i_sd = jax.ShapeDtypeStruct((_B, _H, _W), jnp.int32)
    f_sd = jax.ShapeDtypeStruct((_B, _H, _W), jnp.float32)
    out_spec = pl.BlockSpec((1, _BLK_H, _W), lambda b, h: (b, h, 0))
    return pl.pallas_call(
        _stage_a_body,
        grid=(_B, n_hb),
        in_specs=[pl.BlockSpec((1, 3, _BLK_H, _W), lambda b, h: (b, 0, h, 0))],
        out_specs=[out_spec] * 7,
        out_shape=[i_sd, i_sd, i_sd, f_sd, f_sd, f_sd, f_sd],
    )(warped)


def _stage_b_body(table, i0, i1, g0, g1,
                  i0v, i1v, gv0, gv1, gsem, isem, osem):
    wid = lax.axis_index("s") * 2 + lax.axis_index("c")

    def chunk_body(i, _):
        row0 = wid * (_PPW // 128) + i * _KG
        p0 = wid * _PPW + i * _CH

        h_in = []
        for src_ref, dst in ((i0, i0v), (i1, i1v)):
            h_in.append(pltpu.make_async_copy(
                src_ref.at[pl.ds(row0, _KG), :], dst, isem))
        for h in h_in:
            h.start()
        for h in h_in:
            h.wait()

        h_g = []
        for idx_v, dst in ((i0v, gv0), (i1v, gv1)):
            for k in range(_KG):
                h_g.append(pltpu.make_async_copy(
                    table.at[idx_v.at[k]],
                    dst.at[pl.ds(k * 128, 128), :], gsem))
        for h in h_g:
            h.start()
        for h in h_g:
            h.wait()

        h_out = []
        for gv, out_ref in ((gv0, g0), (gv1, g1)):
            for ch in range(6):
                h_out.append(pltpu.make_async_copy(
                    gv.at[:, pl.ds(ch, 1)],
                    out_ref.at[ch, pl.ds(p0, _CH), :], osem))
        for h in h_out:
            h.start()
        for h in h_out:
            h.wait()
        return 0

    lax.fori_loop(0, _NCHUNK, chunk_body, 0)


def _stage_b(im8, i02, i12):
    mesh = plsc.VectorSubcoreMesh(core_axis_name="c", subcore_axis_name="s")
    g_sd = jax.ShapeDtypeStruct((6, _N, 1), jnp.float32)
    kern = pl.kernel(
        _stage_b_body,
        out_type=(g_sd, g_sd),
        mesh=mesh,
        scratch_types=[
            pltpu.VMEM((_KG, 128), jnp.int32),
            pltpu.VMEM((_KG, 128), jnp.int32),
            pltpu.VMEM((_CH, 8), jnp.float32),
            pltpu.VMEM((_CH, 8), jnp.float32),
            pltpu.SemaphoreType.DMA,
            pltpu.SemaphoreType.DMA,
            pltpu.SemaphoreType.DMA,
        ],
        compiler_params=pltpu.CompilerParams(use_tc_tiling_on_sc=False),
    )
    return kern(im8, i02, i12)


def _stage_c_body(g0_ref, g1_ref, sel_ref,
                  wa_ref, wb_ref, wc_ref, wd_ref, out_ref):
    w_a = wa_ref[0]
    w_b = wb_ref[0]
    w_c = wc_ref[0]
    w_d = wd_ref[0]
    hi = sel_ref[0] > 0
    for ch in range(_C):
        i_a = g0_ref[ch, 0]
        i_b = g1_ref[ch, 0]
        i_c = jnp.where(hi, g0_ref[ch + 3, 0], i_a)
        i_d = jnp.where(hi, g1_ref[ch + 3, 0], i_b)
        out_ref[0, ch] = ((w_a * i_a + w_b * i_b) + w_c * i_c) + w_d * i_d


def _stage_c(g0, g1, sel, wa, wb, wc, wd):
    n_hb = _H // _BLK_H
    g_spec = pl.BlockSpec((6, 1, _BLK_H, _W), lambda b, h: (0, b, h, 0))
    w_spec = pl.BlockSpec((1, _BLK_H, _W), lambda b, h: (b, h, 0))
    return pl.pallas_call(
        _stage_c_body,
        grid=(_B, n_hb),
        in_specs=[g_spec, g_spec] + [w_spec] * 5,
        out_specs=pl.BlockSpec((1, _C, _BLK_H, _W), lambda b, h: (b, 0, h, 0)),
        out_shape=jax.ShapeDtypeStruct((_B, _C, _H, _W), jnp.float32),
    )(g0, g1, sel, wa, wb, wc, wd)


def kernel(src, H):
    b, c, h, w = src.shape
    xx = jnp.tile(jnp.arange(w)[None, :], (h, 1))
    yy = jnp.tile(jnp.arange(h)[:, None], (1, w))
    ones = jnp.ones((h, w), dtype=jnp.int32)
    g = jnp.stack([xx, yy, ones], axis=0).astype(jnp.float32)
    grid = jnp.broadcast_to(g[None], (b, 3, h, w))
    warped = jnp.einsum('bij,bjhw->bihw', H.reshape(b, 3, 3), grid)
    i0, i1, sel, wa, wb, wc, wd = _stage_a(warped)
    im_flat = src.transpose(0, 2, 3, 1).reshape(-1, c)
    shifted = jnp.concatenate([im_flat[1:], im_flat[:1]], axis=0)
    im8 = jnp.concatenate(
        [im_flat, shifted, jnp.zeros((_N, 2), jnp.float32)], axis=1)
    g0, g1 = _stage_b(
        im8, i0.reshape(_N // 128, 128), i1.reshape(_N // 128, 128))
    g0 = g0.reshape(6, _B, _H, _W)
    g1 = g1.reshape(6, _B, _H, _W)
    return _stage_c(g0, g1, sel, wa, wb, wc, wd)


# speedup vs baseline: 12.7218x; 12.7218x over previous
import jax
import jax.numpy as jnp
from jax import lax
from jax.experimental import pallas as pl
from jax.experimental.pallas import tpu as pltpu
from jax.experimental.pallas import tpu_sc as plsc

_B, _C, _H, _W = 16, 3, 512, 512
_N = _B * _H * _W
_BLK_H = 64

_NW = 32
_PPW = _N // _NW
_CH = 2048
_NCHUNK = _PPW // _CH
_KG = _CH // 128


def _stage_a_body(w_ref, i0_ref, i1_ref, sel_ref,
                  wa_ref, wb_ref, wc_ref, wd_ref):
    b = pl.program_id(0)
    hb = pl.program_id(1)
    X = w_ref[0, 0]
    Y = w_ref[0, 1]
    T = w_ref[0, 2]
    xx = lax.broadcasted_iota(jnp.int32, (_BLK_H, _W), 1).astype(jnp.float32)
    yyi = lax.broadcasted_iota(jnp.int32, (_BLK_H, _W), 0) + hb * _BLK_H
    yy = yyi.astype(jnp.float32)
    sm = jnp.where(jnp.abs(T) >= 1e-07, jnp.float32(0.0), jnp.float32(1e-06))
    Tt = T + sm
    v1 = X / Tt
    v2 = Y / Tt
    vgx = xx + (v1 - xx)
    vgy = yy + (v2 - yy)
    x0i = jnp.floor(vgx).astype(jnp.int32)
    y0i = jnp.floor(vgy).astype(jnp.int32)
    x0 = jnp.clip(x0i, 0, _W - 1)
    x1 = jnp.clip(x0i + 1, 0, _W - 1)
    y0 = jnp.clip(y0i, 0, _H - 1)
    y1 = jnp.clip(y0i + 1, 0, _H - 1)
    x0f = x0.astype(jnp.float32)
    x1f = x1.astype(jnp.float32)
    y0f = y0.astype(jnp.float32)
    y1f = y1.astype(jnp.float32)
    Xa = x1f - vgx
    Xc = vgx - x0f
    Ya = y1f - vgy
    Yb = vgy - y0f
    wa_ref[0] = Xa * Ya
    wb_ref[0] = Xa * Yb
    wc_ref[0] = Xc * Ya
    wd_ref[0] = Xc * Yb
    base = b * (_H * _W)
    i0_ref[0] = (base + y0 * _W) + x0
    i1_ref[0] = (base + y1 * _W) + x0
    sel_ref[0] = x1 - x0


def _stage_a(warped):
    n_hb = _H // _BLK_H
    i_sd = jax.ShapeDtypeStruct((_B, _H, _W), jnp.int32)
    f_sd = jax.ShapeDtypeStruct((_B, _H, _W), jnp.float32)
    out_spec = pl.BlockSpec((1, _BLK_H, _W), lambda b, h: (b, h, 0))
    return pl.pallas_call(
        _stage_a_body,
        grid=(_B, n_hb),
        in_specs=[pl.BlockSpec((1, 3, _BLK_H, _W), lambda b, h: (b, 0, h, 0))],
        out_specs=[out_spec] * 7,
        out_shape=[i_sd, i_sd, i_sd, f_sd, f_sd, f_sd, f_sd],
    )(warped)


def _stage_b_body(table, i0, i1, g0, g1,
                  i0v, i1v, gv0, gv1, gsem, isem, osem):
    wid = lax.axis_index("s") * 2 + lax.axis_index("c")

    def chunk_body(i, _):
        row0 = wid * (_PPW // 128) + i * _KG
        p0 = wid * _PPW + i * _CH

        h_in = []
        for src_ref, dst in ((i0, i0v), (i1, i1v)):
            h_in.append(pltpu.make_async_copy(
                src_ref.at[pl.ds(row0, _KG), :], dst, isem))
        for h in h_in:
            h.start()
        for h in h_in:
            h.wait()

        h_g = []
        for idx_v, dst in ((i0v, gv0), (i1v, gv1)):
            for k in range(_KG):
                h_g.append(pltpu.make_async_copy(
                    table.at[idx_v.at[k]],
                    dst.at[pl.ds(k * 128, 128), :], gsem))
        for h in h_g:
            h.start()
        for h in h_g:
            h.wait()

        h_out = []
        for gv, out_ref in ((gv0, g0), (gv1, g1)):
            h_out.append(pltpu.make_async_copy(
                gv, out_ref.at[pl.ds(p0, _CH), :], osem))
        for h in h_out:
            h.start()
        for h in h_out:
            h.wait()
        return 0

    lax.fori_loop(0, _NCHUNK, chunk_body, 0)


def _stage_b(im8, i02, i12):
    mesh = plsc.VectorSubcoreMesh(core_axis_name="c", subcore_axis_name="s")
    g_sd = jax.ShapeDtypeStruct((_N, 8), jnp.float32)
    kern = pl.kernel(
        _stage_b_body,
        out_type=(g_sd, g_sd),
        mesh=mesh,
        scratch_types=[
            pltpu.VMEM((_KG, 128), jnp.int32),
            pltpu.VMEM((_KG, 128), jnp.int32),
            pltpu.VMEM((_CH, 8), jnp.float32),
            pltpu.VMEM((_CH, 8), jnp.float32),
            pltpu.SemaphoreType.DMA,
            pltpu.SemaphoreType.DMA,
            pltpu.SemaphoreType.DMA,
        ],
        compiler_params=pltpu.CompilerParams(use_tc_tiling_on_sc=False),
    )
    return kern(im8, i02, i12)


def _stage_c_body(g0_ref, g1_ref, sel_ref,
                  wa_ref, wb_ref, wc_ref, wd_ref, out_ref):
    w_a = wa_ref[0]
    w_b = wb_ref[0]
    w_c = wc_ref[0]
    w_d = wd_ref[0]
    hi = sel_ref[0] > 0
    for ch in range(_C):
        i_a = g0_ref[ch, 0]
        i_b = g1_ref[ch, 0]
        i_c = jnp.where(hi, g0_ref[ch + 3, 0], i_a)
        i_d = jnp.where(hi, g1_ref[ch + 3, 0], i_b)
        out_ref[0, ch] = ((w_a * i_a + w_b * i_b) + w_c * i_c) + w_d * i_d


def _stage_c(g0, g1, sel, wa, wb, wc, wd):
    n_hb = _H // _BLK_H
    g_spec = pl.BlockSpec((6, 1, _BLK_H, _W), lambda b, h: (0, b, h, 0))
    w_spec = pl.BlockSpec((1, _BLK_H, _W), lambda b, h: (b, h, 0))
    return pl.pallas_call(
        _stage_c_body,
        grid=(_B, n_hb),
        in_specs=[g_spec, g_spec] + [w_spec] * 5,
        out_specs=pl.BlockSpec((1, _C, _BLK_H, _W), lambda b, h: (b, 0, h, 0)),
        out_shape=jax.ShapeDtypeStruct((_B, _C, _H, _W), jnp.float32),
    )(g0, g1, sel, wa, wb, wc, wd)


def kernel(src, H):
    b, c, h, w = src.shape
    xx = jnp.tile(jnp.arange(w)[None, :], (h, 1))
    yy = jnp.tile(jnp.arange(h)[:, None], (1, w))
    ones = jnp.ones((h, w), dtype=jnp.int32)
    g = jnp.stack([xx, yy, ones], axis=0).astype(jnp.float32)
    grid = jnp.broadcast_to(g[None], (b, 3, h, w))
    warped = jnp.einsum('bij,bjhw->bihw', H.reshape(b, 3, 3), grid)
    i0, i1, sel, wa, wb, wc, wd = _stage_a(warped)
    im_flat = src.transpose(0, 2, 3, 1).reshape(-1, c)
    shifted = jnp.concatenate([im_flat[1:], im_flat[:1]], axis=0)
    im8 = jnp.concatenate(
        [im_flat, shifted, jnp.zeros((_N, 2), jnp.float32)], axis=1)
    g0, g1 = _stage_b(
        im8, i0.reshape(_N // 128, 128), i1.reshape(_N // 128, 128))
    g0 = g0[:, :6].T.reshape(6, _B, _H, _W)
    g1 = g1[:, :6].T.reshape(6, _B, _H, _W)
    return _stage_c(g0, g1, sel, wa, wb, wc, wd)
